# baseline (speedup 1.0000x reference)
import functools

import jax
import jax.numpy as jnp
from jax import lax
from jax.experimental import pallas as pl
from jax.experimental.pallas import tpu as pltpu
from jax.experimental.pallas import tpu_sc as plsc

P = 16
CHUNK = 100
NBUF = 8
NACC = 4


def _make_project(V, D, RB):
    def body(t_ref, w_ref, b_ref, o_ref):
        o_ref[...] = lax.dot_general(
            t_ref[...], w_ref[...],
            (((1,), (0,)), ((), ())),
            preferred_element_type=jnp.float32,
        ) + b_ref[...]

    return pl.pallas_call(
        body,
        grid=(V // RB,),
        in_specs=[
            pl.BlockSpec((RB, D), lambda i: (i, 0)),
            pl.BlockSpec((D, P), lambda i: (0, 0)),
            pl.BlockSpec((1, P), lambda i: (0, 0)),
        ],
        out_specs=pl.BlockSpec((RB, P), lambda i: (i, 0)),
        out_shape=jax.ShapeDtypeStruct((V, P), jnp.float32),
    )


def _make_pool(B, L):
    info = plsc.get_sparse_core_info()
    NC, NS = info.num_cores, info.num_subcores
    NW = NC * NS
    assert B % NW == 0 and L % CHUNK == 0 and CHUNK % NACC == 0
    chunks_per_seq = L // CHUNK
    seq_per_w = B // NW
    rows_per_w = seq_per_w * chunks_per_seq
    mesh = plsc.VectorSubcoreMesh(core_axis_name="c", subcore_axis_name="s")

    @functools.partial(
        pl.kernel,
        mesh=mesh,
        compiler_params=pltpu.CompilerParams(use_tc_tiling_on_sc=False),
        out_type=jax.ShapeDtypeStruct((B, P), jnp.float32),
        scratch_types=[
            pltpu.VMEM((rows_per_w, CHUNK), jnp.int32),
            *([pltpu.VMEM((CHUNK, P), jnp.float32)] * NBUF),
            pltpu.VMEM((seq_per_w, P), jnp.float32),
            *([pltpu.SemaphoreType.DMA] * NBUF),
        ],
    )
    def pool(idx_hbm, proj_hbm, out_hbm, idx_v, *rest):
        bufs = rest[:NBUF]
        pooled_v = rest[NBUF]
        sems = rest[NBUF + 1:]
        wid = lax.axis_index("s") * NC + lax.axis_index("c")

        pltpu.sync_copy(idx_hbm.at[pl.ds(wid * rows_per_w, rows_per_w)], idx_v)

        def gather(c, i):
            return pltpu.make_async_copy(
                proj_hbm.at[idx_v.at[c]], bufs[i], sems[i])

        for c in range(NBUF - 1):
            gather(c, c).start()

        def accum(buf, a):
            def body(j, a):
                return tuple(
                    a[k] + buf[NACC * j + k, pl.ds(0, P)] for k in range(NACC))
            return lax.fori_loop(0, CHUNK // NACC, body, a, unroll=4)

        zero = jnp.zeros((P,), jnp.float32)

        def outer_body(ss, carry):
            for so in range(NBUF // chunks_per_seq):
                s = (NBUF // chunks_per_seq) * ss + so
                a = (zero,) * NACC
                for i in range(chunks_per_seq):
                    u = chunks_per_seq * so + i
                    c = NBUF * ss + u
                    gather(c, u).wait()

                    @pl.when(c + NBUF - 1 < rows_per_w)
                    def _start_next():
                        gather(c + NBUF - 1, (u + NBUF - 1) % NBUF).start()

                    a = accum(bufs[u], a)
                pooled_v[s, pl.ds(0, P)] = (a[0] + a[1]) + (a[2] + a[3])
            return carry

        lax.fori_loop(0, rows_per_w // NBUF, outer_body, 0)
        pltpu.sync_copy(pooled_v, out_hbm.at[pl.ds(wid * seq_per_w, seq_per_w)])

    return pool


def kernel(tinputs, tinputs_len, table, W, b):
    B, L = tinputs.shape
    V, D = table.shape
    OUT = W.shape[0]
    Wp = jnp.zeros((D, P), jnp.float32).at[:, :OUT].set(W.T)
    bp = jnp.zeros((1, P), jnp.float32).at[0, :OUT].set(b / L)
    proj = _make_project(V, D, 8000)(table, Wp, bp)
    idx = tinputs.astype(jnp.int32).reshape(B * L // CHUNK, CHUNK)
    pooled = _make_pool(B, L)(idx, proj)
    return pooled[:, :OUT]


# speedup vs baseline: 1.2728x; 1.2728x over previous
import functools

import jax
import jax.numpy as jnp
from jax import lax
from jax.experimental import pallas as pl
from jax.experimental.pallas import tpu as pltpu
from jax.experimental.pallas import tpu_sc as plsc

P = 16
CHUNK = 100
NBUF = 8
NACC = 4


def _make_project(V8, D8, RB):
    def body(t_ref, w_ref, b_ref, o_ref):
        o_ref[...] = lax.dot_general(
            t_ref[...], w_ref[...],
            (((1,), (0,)), ((), ())),
            preferred_element_type=jnp.float32,
        ) + b_ref[...]

    return pl.pallas_call(
        body,
        grid=(V8 // RB,),
        in_specs=[
            pl.BlockSpec((RB, D8), lambda i: (i, 0)),
            pl.BlockSpec((D8, 8 * P), lambda i: (0, 0)),
            pl.BlockSpec((1, 8 * P), lambda i: (0, 0)),
        ],
        out_specs=pl.BlockSpec((RB, 8 * P), lambda i: (i, 0)),
        out_shape=jax.ShapeDtypeStruct((V8, 8 * P), jnp.float32),
    )


def _make_pool(B, L):
    info = plsc.get_sparse_core_info()
    NC, NS = info.num_cores, info.num_subcores
    NW = NC * NS
    assert B % NW == 0 and L % CHUNK == 0 and CHUNK % NACC == 0
    chunks_per_seq = L // CHUNK
    seq_per_w = B // NW
    rows_per_w = seq_per_w * chunks_per_seq
    mesh = plsc.VectorSubcoreMesh(core_axis_name="c", subcore_axis_name="s")

    @functools.partial(
        pl.kernel,
        mesh=mesh,
        compiler_params=pltpu.CompilerParams(use_tc_tiling_on_sc=False),
        out_type=jax.ShapeDtypeStruct((B, P), jnp.float32),
        scratch_types=[
            pltpu.VMEM((rows_per_w, CHUNK), jnp.int32),
            *([pltpu.VMEM((CHUNK, P), jnp.float32)] * NBUF),
            pltpu.VMEM((seq_per_w, P), jnp.float32),
            *([pltpu.SemaphoreType.DMA] * NBUF),
        ],
    )
    def pool(idx_hbm, proj_hbm, out_hbm, idx_v, *rest):
        bufs = rest[:NBUF]
        pooled_v = rest[NBUF]
        sems = rest[NBUF + 1:]
        wid = lax.axis_index("s") * NC + lax.axis_index("c")

        pltpu.sync_copy(idx_hbm.at[pl.ds(wid * rows_per_w, rows_per_w)], idx_v)

        def gather(c, i):
            return pltpu.make_async_copy(
                proj_hbm.at[idx_v.at[c]], bufs[i], sems[i])

        for c in range(NBUF - 1):
            gather(c, c).start()

        def accum(buf, a):
            def body(j, a):
                return tuple(
                    a[k] + buf[NACC * j + k, pl.ds(0, P)] for k in range(NACC))
            return lax.fori_loop(0, CHUNK // NACC, body, a, unroll=4)

        zero = jnp.zeros((P,), jnp.float32)

        def outer_body(ss, carry):
            for so in range(NBUF // chunks_per_seq):
                s = (NBUF // chunks_per_seq) * ss + so
                a = (zero,) * NACC
                for i in range(chunks_per_seq):
                    u = chunks_per_seq * so + i
                    c = NBUF * ss + u
                    gather(c, u).wait()

                    @pl.when(c + NBUF - 1 < rows_per_w)
                    def _start_next():
                        gather(c + NBUF - 1, (u + NBUF - 1) % NBUF).start()

                    a = accum(bufs[u], a)
                pooled_v[s, pl.ds(0, P)] = (a[0] + a[1]) + (a[2] + a[3])
            return carry

        lax.fori_loop(0, rows_per_w // NBUF, outer_body, 0)
        pltpu.sync_copy(pooled_v, out_hbm.at[pl.ds(wid * seq_per_w, seq_per_w)])

    return pool


def kernel(tinputs, tinputs_len, table, W, b):
    B, L = tinputs.shape
    V, D = table.shape
    OUT = W.shape[0]
    Wp = jnp.zeros((D, P), jnp.float32).at[:, :OUT].set(W.T)
    bp = jnp.zeros((P,), jnp.float32).at[:OUT].set(b / L)
    Wp_big = jax.scipy.linalg.block_diag(*([Wp] * 8))
    bp_big = jnp.tile(bp, 8).reshape(1, 8 * P)
    table_r = table.reshape(V // 8, 8 * D)
    proj = _make_project(V // 8, 8 * D, 1000)(table_r, Wp_big, bp_big)
    idx = tinputs.astype(jnp.int32).reshape(B * L // CHUNK, CHUNK)
    pooled = _make_pool(B, L)(idx, proj.reshape(V, P))
    return pooled[:, :OUT]


# speedup vs baseline: 1.5282x; 1.2006x over previous
import functools

import jax
import jax.numpy as jnp
from jax import lax
from jax.experimental import pallas as pl
from jax.experimental.pallas import tpu as pltpu
from jax.experimental.pallas import tpu_sc as plsc

P = 16
CHUNK = 100
NBUF = 8
NACC = 4


def _make_project(V8, D8, RB):
    D = D8 // 8

    def body(t_ref, w_ref, b_ref, o_ref):
        z = lax.dot_general(
            t_ref[...], w_ref[...],
            (((1,), (0,)), ((), ())),
            preferred_element_type=jnp.float32,
        )
        r = lax.broadcasted_iota(jnp.int32, (8 * RB, 8 * P), 0)
        c = lax.broadcasted_iota(jnp.int32, (8 * RB, 8 * P), 1)
        y = jnp.where((c // P) == (r % 8), z, 0.0)
        o_ref[...] = y.reshape(RB, 8, 8 * P).sum(axis=1) + b_ref[...]

    return pl.pallas_call(
        body,
        grid=(V8 // RB,),
        in_specs=[
            pl.BlockSpec((8 * RB, D), lambda i: (i, 0)),
            pl.BlockSpec((D, 8 * P), lambda i: (0, 0)),
            pl.BlockSpec((1, 8 * P), lambda i: (0, 0)),
        ],
        out_specs=pl.BlockSpec((RB, 8 * P), lambda i: (i, 0)),
        out_shape=jax.ShapeDtypeStruct((V8, 8 * P), jnp.float32),
    )


def _make_pool(B, L):
    info = plsc.get_sparse_core_info()
    NC, NS = info.num_cores, info.num_subcores
    NW = NC * NS
    assert B % NW == 0 and L % CHUNK == 0 and CHUNK % NACC == 0
    chunks_per_seq = L // CHUNK
    seq_per_w = B // NW
    rows_per_w = seq_per_w * chunks_per_seq
    mesh = plsc.VectorSubcoreMesh(core_axis_name="c", subcore_axis_name="s")

    @functools.partial(
        pl.kernel,
        mesh=mesh,
        compiler_params=pltpu.CompilerParams(use_tc_tiling_on_sc=False),
        out_type=jax.ShapeDtypeStruct((B, P), jnp.float32),
        scratch_types=[
            pltpu.VMEM((rows_per_w, CHUNK), jnp.int32),
            *([pltpu.VMEM((CHUNK, P), jnp.float32)] * NBUF),
            pltpu.VMEM((seq_per_w, P), jnp.float32),
            *([pltpu.SemaphoreType.DMA] * NBUF),
        ],
    )
    def pool(idx_hbm, proj_hbm, out_hbm, idx_v, *rest):
        bufs = rest[:NBUF]
        pooled_v = rest[NBUF]
        sems = rest[NBUF + 1:]
        wid = lax.axis_index("s") * NC + lax.axis_index("c")

        pltpu.sync_copy(idx_hbm.at[pl.ds(wid * rows_per_w, rows_per_w)], idx_v)

        def gather(c, i):
            return pltpu.make_async_copy(
                proj_hbm.at[idx_v.at[c]], bufs[i], sems[i])

        for c in range(NBUF - 1):
            gather(c, c).start()

        def accum(buf, a):
            def body(j, a):
                return tuple(
                    a[k] + buf[NACC * j + k, pl.ds(0, P)] for k in range(NACC))
            return lax.fori_loop(0, CHUNK // NACC, body, a, unroll=4)

        zero = jnp.zeros((P,), jnp.float32)

        def outer_body(ss, carry):
            for so in range(NBUF // chunks_per_seq):
                s = (NBUF // chunks_per_seq) * ss + so
                a = (zero,) * NACC
                for i in range(chunks_per_seq):
                    u = chunks_per_seq * so + i
                    c = NBUF * ss + u
                    gather(c, u).wait()

                    @pl.when(c + NBUF - 1 < rows_per_w)
                    def _start_next():
                        gather(c + NBUF - 1, (u + NBUF - 1) % NBUF).start()

                    a = accum(bufs[u], a)
                pooled_v[s, pl.ds(0, P)] = (a[0] + a[1]) + (a[2] + a[3])
            return carry

        lax.fori_loop(0, rows_per_w // NBUF, outer_body, 0)
        pltpu.sync_copy(pooled_v, out_hbm.at[pl.ds(wid * seq_per_w, seq_per_w)])

    return pool


def kernel(tinputs, tinputs_len, table, W, b):
    B, L = tinputs.shape
    V, D = table.shape
    OUT = W.shape[0]
    Wp = jnp.zeros((D, P), jnp.float32).at[:, :OUT].set(W.T)
    bp = jnp.zeros((P,), jnp.float32).at[:OUT].set(b / L)
    Wp_wide = jnp.tile(Wp, (1, 8))
    bp_wide = jnp.tile(bp, 8).reshape(1, 8 * P)
    proj = _make_project(V // 8, 8 * D, 1000)(table, Wp_wide, bp_wide)
    idx = tinputs.astype(jnp.int32).reshape(B * L // CHUNK, CHUNK)
    pooled = _make_pool(B, L)(idx, proj.reshape(V, P))
    return pooled[:, :OUT]


# speedup vs baseline: 1.6561x; 1.0837x over previous
import functools

import jax
import jax.numpy as jnp
from jax import lax
from jax.experimental import pallas as pl
from jax.experimental.pallas import tpu as pltpu
from jax.experimental.pallas import tpu_sc as plsc

P = 16
CHUNK = 100
NBUF = 8
NACC = 4


def _make_project(V8, D8, RB):
    D = D8 // 8

    def body(t_ref, w_ref, b_ref, o_ref):
        z = lax.dot_general(
            t_ref[...], w_ref[...],
            (((1,), (0,)), ((), ())),
            preferred_element_type=jnp.float32,
        )
        r = lax.broadcasted_iota(jnp.int32, (8 * RB, 8 * P), 0)
        c = lax.broadcasted_iota(jnp.int32, (8 * RB, 8 * P), 1)
        y = jnp.where((c // P) == (r % 8), z, 0.0)
        o_ref[...] = y.reshape(RB, 8, 8 * P).sum(axis=1) + b_ref[...]

    return pl.pallas_call(
        body,
        grid=(V8 // RB,),
        in_specs=[
            pl.BlockSpec((8 * RB, D), lambda i: (i, 0)),
            pl.BlockSpec((D, 8 * P), lambda i: (0, 0)),
            pl.BlockSpec((1, 8 * P), lambda i: (0, 0)),
        ],
        out_specs=pl.BlockSpec((RB, 8 * P), lambda i: (i, 0)),
        out_shape=jax.ShapeDtypeStruct((V8, 8 * P), jnp.float32),
    )


def _make_pool(B, L):
    info = plsc.get_sparse_core_info()
    NC, NS = info.num_cores, info.num_subcores
    NW = NC * NS
    assert B % NW == 0 and L % CHUNK == 0 and CHUNK % NACC == 0
    chunks_per_seq = L // CHUNK
    seq_per_w = B // NW
    rows_per_w = seq_per_w * chunks_per_seq
    mesh = plsc.VectorSubcoreMesh(core_axis_name="c", subcore_axis_name="s")

    @functools.partial(
        pl.kernel,
        mesh=mesh,
        compiler_params=pltpu.CompilerParams(use_tc_tiling_on_sc=False),
        out_type=jax.ShapeDtypeStruct((B, P), jnp.float32),
        scratch_types=[
            pltpu.VMEM((rows_per_w, CHUNK), jnp.int32),
            *([pltpu.VMEM((CHUNK, P), jnp.float32)] * NBUF),
            pltpu.VMEM((seq_per_w, P), jnp.float32),
            *([pltpu.SemaphoreType.DMA] * NBUF),
        ],
    )
    def pool(idx_hbm, proj_hbm, out_hbm, idx_v, *rest):
        bufs = rest[:NBUF]
        pooled_v = rest[NBUF]
        sems = rest[NBUF + 1:]
        wid = lax.axis_index("s") * NC + lax.axis_index("c")

        pltpu.sync_copy(idx_hbm.at[pl.ds(wid * rows_per_w, rows_per_w)], idx_v)

        def gather(c, i):
            return pltpu.make_async_copy(
                proj_hbm.at[idx_v.at[c]], bufs[i], sems[i])

        for c in range(NBUF - 1):
            gather(c, c).start()

        def accum(buf, a):
            def body(j, a):
                return tuple(
                    a[k] + buf[NACC * j + k, pl.ds(0, P)] for k in range(NACC))
            return lax.fori_loop(0, CHUNK // NACC, body, a, unroll=4)

        zero = jnp.zeros((P,), jnp.float32)

        def outer_body(ss, carry):
            for so in range(NBUF // chunks_per_seq):
                s = (NBUF // chunks_per_seq) * ss + so
                a = (zero,) * NACC
                for i in range(chunks_per_seq):
                    u = chunks_per_seq * so + i
                    c = NBUF * ss + u
                    gather(c, u).wait()

                    @pl.when(c + NBUF - 1 < rows_per_w)
                    def _start_next():
                        gather(c + NBUF - 1, (u + NBUF - 1) % NBUF).start()

                    a = accum(bufs[u], a)
                pooled_v[s, pl.ds(0, P)] = (a[0] + a[1]) + (a[2] + a[3])
            return carry

        lax.fori_loop(0, rows_per_w // NBUF, outer_body, 0)
        pltpu.sync_copy(pooled_v, out_hbm.at[pl.ds(wid * seq_per_w, seq_per_w)])

    return pool


def kernel(tinputs, tinputs_len, table, W, b):
    B, L = tinputs.shape
    V, D = table.shape
    OUT = W.shape[0]
    Wp = jnp.zeros((D, P), jnp.float32).at[:, :OUT].set(W.T)
    bp = jnp.zeros((P,), jnp.float32).at[:OUT].set(b / L)
    Wp_wide = jnp.tile(Wp, (1, 8))
    bp_wide = jnp.tile(bp, 8).reshape(1, 8 * P)
    proj = _make_project(V // 8, 8 * D, 5000)(table, Wp_wide, bp_wide)
    idx = tinputs.astype(jnp.int32).reshape(B * L // CHUNK, CHUNK)
    pooled = _make_pool(B, L)(idx, proj.reshape(V, P))
    return pooled[:, :OUT]
